# Initial kernel scaffold; baseline (speedup 1.0000x reference)
import jax
import jax.numpy as jnp
from jax.experimental import pallas as pl


def kernel(s, t):
    raise NotImplementedError("write your pallas kernel here")


# speedup vs baseline: 4.6996x; 4.6996x over previous
import jax
import jax.numpy as jnp
from jax import lax
from jax.experimental import pallas as pl

_H = 32
_W = 32
_C = 192
_HW = _H * _W


def _shifted_windows(x):
    xp = jnp.pad(x[0], ((0, 0), (1, 1), (1, 1)), mode="edge")
    xp = jnp.transpose(xp, (1, 2, 0))
    return jnp.stack([xp[:, j:j + _W, :] for j in range(3)], axis=0)


def _patch_match_kernel(s_ref, t_ref, out_ref):
    acc = jnp.zeros((_HW, _HW), dtype=jnp.float32)
    qsq = jnp.zeros((_HW,), dtype=jnp.float32)
    psq = jnp.zeros((_HW,), dtype=jnp.float32)
    for j in range(3):
        for i in range(3):
            sblk = s_ref[j, i:i + _H].reshape(_HW, _C)
            tblk = t_ref[j, i:i + _H].reshape(_HW, _C)
            acc = acc + lax.dot_general(
                sblk, tblk,
                dimension_numbers=(((1,), (1,)), ((), ())),
                preferred_element_type=jnp.float32,
            )
            qsq = qsq + jnp.sum(sblk * sblk, axis=1)
            psq = psq + jnp.sum(tblk * tblk, axis=1)
    d2 = qsq[:, None] - 2.0 * acc + psq[None, :]
    rid = jnp.argmin(d2, axis=1).astype(jnp.int32)
    out_ref[0, :] = rid // _W
    out_ref[1, :] = rid % _W


def kernel(s, t):
    n = s.shape[0]
    sw = _shifted_windows(s)
    tw = _shifted_windows(t)
    out = pl.pallas_call(
        _patch_match_kernel,
        out_shape=jax.ShapeDtypeStruct((2, _HW), jnp.int32),
    )(sw, tw)
    return out.reshape(n, 2, _H, _W)
